# Initial kernel scaffold; baseline (speedup 1.0000x reference)
import jax
import jax.numpy as jnp
from jax.experimental import pallas as pl


def kernel(x, neighbor_indices, Wl0, bl0, Wl1, bl1, Wl2, bl2, ln0_g, ln0_b, ln1_g, ln1_b):
    raise NotImplementedError("write your pallas kernel here")


# speedup vs baseline: 4.7238x; 4.7238x over previous
import functools

import jax
import jax.numpy as jnp
from jax import lax
from jax.experimental import pallas as pl
from jax.experimental.pallas import tpu as pltpu
from jax.experimental.pallas import tpu_sc as plsc

N = 50000
D = 128
K = 10

NC = 2
NS = 16
NW = NC * NS

BN = 8
STEPS = 196
C_PER_W = BN * STEPS
NPAD = NW * C_PER_W

_mesh = plsc.VectorSubcoreMesh(core_axis_name="c", subcore_axis_name="s")


@functools.partial(
    pl.kernel,
    out_type=jax.ShapeDtypeStruct((NPAD, D), jnp.float32),
    mesh=_mesh,
    scratch_types=[
        pltpu.VMEM((STEPS, BN * K), jnp.int32),
        pltpu.VMEM((2, BN * K, D), jnp.float32),
        pltpu.VMEM((2, BN, D), jnp.float32),
        pltpu.SemaphoreType.DMA,
        pltpu.SemaphoreType.DMA,
        pltpu.SemaphoreType.DMA,
        pltpu.SemaphoreType.DMA,
    ],
)
def _neighbor_sum(h_hbm, idx_hbm, out_hbm, idx_v, rows_v, out_v,
                  si0, si1, so0, so1):
    c = lax.axis_index("c")
    s = lax.axis_index("s")
    wid = s * NC + c
    base = wid * C_PER_W
    sin = (si0, si1)
    sout = (so0, so1)

    pltpu.sync_copy(idx_hbm.at[wid], idx_v)
    pltpu.async_copy(h_hbm.at[idx_v.at[0]], rows_v.at[0], sin[0])

    @pl.loop(0, STEPS, step=2)
    def _steps(g):
        for b in range(2):
            gb = g + b
            nb = (b + 1) % 2

            @pl.when(gb + 1 < STEPS)
            def _():
                pltpu.async_copy(h_hbm.at[idx_v.at[gb + 1]], rows_v.at[nb],
                                 sin[nb])

            pltpu.make_async_copy(h_hbm.at[idx_v.at[gb]], rows_v.at[b],
                                  sin[b]).wait()

            @pl.when(gb >= 2)
            def _():
                pltpu.make_async_copy(
                    out_v.at[b],
                    out_hbm.at[pl.ds(base + (gb - 2) * BN, BN)],
                    sout[b]).wait()

            for nn in range(BN):
                r0 = nn * K
                for gr in range(D // 16):
                    sl = pl.ds(gr * 16, 16)
                    acc = rows_v[b, r0, sl]
                    for k in range(1, K):
                        acc = acc + rows_v[b, r0 + k, sl]
                    out_v[b, nn, sl] = acc

            pltpu.async_copy(out_v.at[b],
                             out_hbm.at[pl.ds(base + gb * BN, BN)],
                             sout[b])

    for b in range(2):
        gb = STEPS - 2 + b
        pltpu.make_async_copy(out_v.at[b],
                              out_hbm.at[pl.ds(base + gb * BN, BN)],
                              sout[b]).wait()


def _tc_body(apply_ln, h_ref, a_ref, ws_ref, wn_ref, b_ref, g_ref, bb_ref,
             o_ref):
    y = jnp.dot(h_ref[...], ws_ref[...], preferred_element_type=jnp.float32)
    y = y + jnp.dot(a_ref[...], wn_ref[...],
                    preferred_element_type=jnp.float32)
    y = y + b_ref[...]
    if apply_ln:
        mu = jnp.mean(y, axis=-1, keepdims=True)
        var = jnp.mean((y - mu) * (y - mu), axis=-1, keepdims=True)
        ln = (y - mu) * lax.rsqrt(var + 1e-5) * g_ref[...] + bb_ref[...]
        y = y + jnp.maximum(ln, 0.0)
    o_ref[...] = y


_RB = 2000
_GRID = N // _RB


def _tc_layer(h, aggsum, ws, wn, bias, g, bb, apply_ln):
    row_spec = pl.BlockSpec((_RB, D), lambda i: (i, 0))
    full_spec = pl.BlockSpec((D, D), lambda i: (0, 0))
    vec_spec = pl.BlockSpec((1, D), lambda i: (0, 0))
    return pl.pallas_call(
        functools.partial(_tc_body, apply_ln),
        grid=(_GRID,),
        in_specs=[row_spec, row_spec, full_spec, full_spec, vec_spec,
                  vec_spec, vec_spec],
        out_specs=row_spec,
        out_shape=jax.ShapeDtypeStruct((N, D), jnp.float32),
    )(h, aggsum, ws, wn, bias, g, bb)


def kernel(x, neighbor_indices, Wl0, bl0, Wl1, bl1, Wl2, bl2,
           ln0_g, ln0_b, ln1_g, ln1_b):
    idx = jnp.pad(neighbor_indices, ((0, NPAD - N), (0, 0)))
    idx = idx.reshape(NW, STEPS, BN * K)

    Wl = [Wl0, Wl1, Wl2]
    bl = [bl0, bl1, bl2]
    ln_g = [ln0_g, ln1_g]
    ln_b = [ln0_b, ln1_b]

    one = jnp.ones((1, D), jnp.float32)
    zero = jnp.zeros((1, D), jnp.float32)

    h = x
    for i in range(3):
        ws = Wl[i][:, :D].T
        wn = Wl[i][:, D:].T * (1.0 / K)
        bias = bl[i].reshape(1, D)
        aggsum = _neighbor_sum(h, idx)[:N]
        if i < 2:
            h = _tc_layer(h, aggsum, ws, wn, bias, ln_g[i].reshape(1, D),
                          ln_b[i].reshape(1, D), True)
        else:
            h = _tc_layer(h, aggsum, ws, wn, bias, one, zero, False)
    return h
